# Initial kernel scaffold; baseline (speedup 1.0000x reference)
import jax
import jax.numpy as jnp
from jax.experimental import pallas as pl


def kernel(x, edge_index, Ws0, bs0, Wn0, bn0, Ws1, bs1, Wn1, bn1, Ws2, bs2, Wn2, bn2):
    raise NotImplementedError("write your pallas kernel here")


# speedup vs baseline: 5.8080x; 5.8080x over previous
import jax
import jax.numpy as jnp
from jax import lax
from jax.experimental import pallas as pl
from jax.experimental.pallas import tpu as pltpu
from jax.experimental.pallas import tpu_sc as plsc

N = 10000
E = 160000
NP = 10240
EP = 163840
K = 128
NC = 2
NS = 16
HALF = NP // 2
AH = HALF + 128
ZPT = AH // NS
WPT = HALF // NS


def _make_agg(two_chunks: bool):
    nc = EP // NS // K
    assert nc % 2 == 0
    nq = 2 if two_chunks else 1

    scratch = [
        pltpu.VMEM((nc, K), jnp.int32),
        pltpu.VMEM((2, nc, K), jnp.int32),
        pltpu.VMEM((K, 128), jnp.float32),
        pltpu.VMEM((K, 128), jnp.float32),
        pltpu.VMEM_SHARED((AH, 128), jnp.float32),
        pltpu.SemaphoreType.DMA,
        pltpu.SemaphoreType.DMA,
    ]
    mesh = plsc.VectorSubcoreMesh(core_axis_name="c", subcore_axis_name="s")

    def body(m_hbm, src_hbm, dst_hbm, zero_hbm, out_hbm,
             idxs, idxd, b0, b1, acc, sem0, sem1):
        c = lax.axis_index("c")
        s = lax.axis_index("s")

        if two_chunks:
            pltpu.sync_copy(src_hbm.at[c * NS + s], idxs)
            pltpu.sync_copy(dst_hbm.at[0].at[s], idxd.at[0])
            pltpu.sync_copy(dst_hbm.at[1].at[s], idxd.at[1])
        else:
            pltpu.sync_copy(src_hbm.at[s], idxs)
            pltpu.sync_copy(dst_hbm.at[c].at[s], idxd.at[0])

        for r in range(2 if two_chunks else 1):
            ixd = idxd.at[r]
            q = c if two_chunks else 0
            h = r if two_chunks else c

            pltpu.sync_copy(zero_hbm.at[pl.ds(s * ZPT, ZPT)],
                            acc.at[pl.ds(s * ZPT, ZPT)])
            plsc.subcore_barrier()

            pltpu.async_copy(m_hbm.at[idxs.at[0]], b0, sem0)

            def step(i, carry):
                j0 = 2 * i
                j1 = 2 * i + 1
                pltpu.async_copy(m_hbm.at[idxs.at[j1]], b1, sem1)
                pltpu.make_async_copy(m_hbm.at[idxs.at[j0]], b0, sem0).wait()
                pltpu.sync_copy(b0, acc.at[ixd.at[j0]], add=True)

                @pl.when(j1 + 1 < nc)
                def _():
                    pltpu.async_copy(m_hbm.at[idxs.at[j1 + 1]], b0, sem0)
                pltpu.make_async_copy(m_hbm.at[idxs.at[j1]], b1, sem1).wait()
                pltpu.sync_copy(b1, acc.at[ixd.at[j1]], add=True)
                return carry

            lax.fori_loop(0, nc // 2, step, 0)
            plsc.subcore_barrier()

            pltpu.sync_copy(
                acc.at[pl.ds(s * WPT, WPT)],
                out_hbm.at[q].at[pl.ds(h * HALF + s * WPT, WPT)])
            if two_chunks and r == 0:
                plsc.subcore_barrier()

    return pl.kernel(body,
                     out_type=jax.ShapeDtypeStruct((nq, NP, 128), jnp.float32),
                     mesh=mesh, scratch_types=tuple(scratch))


def _make_deg():
    nc = EP // NS // K
    scratch = [
        pltpu.VMEM((nc, K), jnp.int32),
        pltpu.VMEM((K, 128), jnp.float32),
        pltpu.VMEM_SHARED((AH, 128), jnp.float32),
    ]
    mesh = plsc.VectorSubcoreMesh(core_axis_name="c", subcore_axis_name="s")

    def body(dst_hbm, ones_hbm, zero_hbm, deg_hbm, idxd, onesb, dacc):
        c = lax.axis_index("c")
        s = lax.axis_index("s")
        pltpu.sync_copy(dst_hbm.at[c].at[s], idxd)
        pltpu.sync_copy(ones_hbm, onesb)
        pltpu.sync_copy(zero_hbm.at[pl.ds(s * ZPT, ZPT)],
                        dacc.at[pl.ds(s * ZPT, ZPT)])
        plsc.subcore_barrier()

        def step(j, carry):
            pltpu.sync_copy(onesb, dacc.at[idxd.at[j]], add=True)
            return carry

        lax.fori_loop(0, nc, step, 0)
        plsc.subcore_barrier()
        pltpu.sync_copy(dacc.at[pl.ds(s * WPT, WPT)],
                        deg_hbm.at[pl.ds(c * HALF + s * WPT, WPT)])

    return pl.kernel(body,
                     out_type=jax.ShapeDtypeStruct((NP, 128), jnp.float32),
                     mesh=mesh, scratch_types=tuple(scratch))


def _dotT(a, w):
    return lax.dot_general(a, w, (((1,), (1,)), ((), ())),
                           preferred_element_type=jnp.float32)


def _mm1_body(x_ref, s_ref, ws_ref, wn_ref, b_ref, inv_ref, o_ref):
    ws = ws_ref[...]
    wn = wn_ref[...]
    inv = inv_ref[...]
    o = (_dotT(x_ref[0], ws[:, :128]) + _dotT(x_ref[1], ws[:, 128:])
         + _dotT(s_ref[0] * inv, wn[:, :128])
         + _dotT(s_ref[1] * inv, wn[:, 128:])
         + b_ref[0])
    o_ref[0] = jnp.maximum(o, 0.0)


def _mm2_body(a_ref, s_ref, ws_ref, wn_ref, b_ref, wpz_ref, wpr_ref, inv_ref,
              z_ref, r_ref, p_ref):
    c = pl.program_id(1)
    ws = ws_ref[...]
    wn = wn_ref[...]
    inv = inv_ref[...]
    z = (_dotT(a_ref[0], ws[:, :128]) + _dotT(a_ref[1], ws[:, 128:])
         + _dotT(s_ref[0] * inv, wn[:, :128])
         + _dotT(s_ref[1] * inv, wn[:, 128:])
         + b_ref[0])
    r = jnp.maximum(z, 0.0)
    z_ref[0] = z
    r_ref[0] = r
    contrib = _dotT(z, wpz_ref[...]) + _dotT(r, wpr_ref[...])

    @pl.when(c == 0)
    def _():
        p_ref[...] = contrib

    @pl.when(c != 0)
    def _():
        p_ref[...] = p_ref[...] + contrib


def _mm3_body(z_ref, r_ref, sp_ref, ws_ref, b_ref, inv_ref, o_ref):
    ws = ws_ref[...]
    inv = inv_ref[...]
    o_ref[...] = (_dotT(z_ref[0], ws[:, 0:128])
                  + _dotT(z_ref[1], ws[:, 128:256])
                  + _dotT(r_ref[0], ws[:, 256:384])
                  + _dotT(r_ref[1], ws[:, 384:512])
                  + sp_ref[0] * inv + b_ref[0])


def kernel(x, edge_index, Ws0, bs0, Wn0, bn0, Ws1, bs1, Wn1, bn1,
           Ws2, bs2, Wn2, bn2):
    f32 = jnp.float32
    src = edge_index[0].astype(jnp.int32)
    dst = edge_index[1].astype(jnp.int32)

    pad = EP - E
    pad_src = (jnp.arange(pad, dtype=jnp.int32) * 64) % N
    src_p = jnp.concatenate([src, pad_src])
    dst_p = jnp.concatenate([dst, jnp.full((pad,), N, jnp.int32)])

    ept = EP // NS
    nc = ept // K

    trash = HALF + (dst_p % 128)
    dst_loc = []
    for h in range(2):
        lo = h * HALF
        in_h = (dst_p >= lo) & (dst_p < lo + HALF)
        dst_loc.append(jnp.where(in_h, dst_p - lo, trash))
    dst2 = jnp.stack(dst_loc).reshape(2, NS, nc, K)

    src_a = jnp.concatenate([src_p, src_p + NP]).reshape(NC * NS, nc, K)
    src_1 = src_p.reshape(NS, nc, K)

    zeroA = jnp.zeros((AH, 128), f32)
    onesK = jnp.ones((K, 128), f32)

    xp = jnp.zeros((NP, 256), f32).at[:N].set(x)
    x_st = jnp.stack([xp[:, :128], xp[:, 128:]])

    agg2c = _make_agg(True)
    agg1c = _make_agg(False)
    deg_kernel = _make_deg()

    degm = deg_kernel(dst2, onesK, zeroA)
    s_x = agg2c(x_st.reshape(NC * NP, 128), src_a, dst2, zeroA)
    inv = (1.0 / jnp.clip(degm[:, 0], 1.0, None)).reshape(NP, 1)

    bm = 1024
    n_i = NP // bm
    grid2 = (n_i, NC)

    def rs(shape, f=f32):
        return jax.ShapeDtypeStruct(shape, f)

    a_st = pl.pallas_call(
        _mm1_body,
        grid=grid2,
        in_specs=[
            pl.BlockSpec((NC, bm, 128), lambda i, c: (0, i, 0)),
            pl.BlockSpec((NC, bm, 128), lambda i, c: (0, i, 0)),
            pl.BlockSpec((128, 256), lambda i, c: (c, 0)),
            pl.BlockSpec((128, 256), lambda i, c: (c, 0)),
            pl.BlockSpec((1, 1, 128), lambda i, c: (c, 0, 0)),
            pl.BlockSpec((bm, 1), lambda i, c: (i, 0)),
        ],
        out_specs=pl.BlockSpec((1, bm, 128), lambda i, c: (c, i, 0)),
        out_shape=rs((NC, NP, 128)),
    )(x_st, s_x, Ws0, Wn0, (bs0 + bn0).reshape(2, 1, 128), inv)

    s_a = agg2c(a_st.reshape(NC * NP, 128), src_a, dst2, zeroA)

    z_st, r_st, p2 = pl.pallas_call(
        _mm2_body,
        grid=grid2,
        in_specs=[
            pl.BlockSpec((NC, bm, 128), lambda i, c: (0, i, 0)),
            pl.BlockSpec((NC, bm, 128), lambda i, c: (0, i, 0)),
            pl.BlockSpec((128, 256), lambda i, c: (c, 0)),
            pl.BlockSpec((128, 256), lambda i, c: (c, 0)),
            pl.BlockSpec((1, 1, 128), lambda i, c: (c, 0, 0)),
            pl.BlockSpec((128, 128), lambda i, c: (0, c)),
            pl.BlockSpec((128, 128), lambda i, c: (0, 2 + c)),
            pl.BlockSpec((bm, 1), lambda i, c: (i, 0)),
        ],
        out_specs=[
            pl.BlockSpec((1, bm, 128), lambda i, c: (c, i, 0)),
            pl.BlockSpec((1, bm, 128), lambda i, c: (c, i, 0)),
            pl.BlockSpec((bm, 128), lambda i, c: (i, 0)),
        ],
        out_shape=[rs((NC, NP, 128)), rs((NC, NP, 128)), rs((NP, 128))],
    )(a_st, s_a, Ws1, Wn1, (bs1 + bn1).reshape(2, 1, 128), Wn2, Wn2, inv)

    s_p = agg1c(p2, src_1, dst2, zeroA)

    out = pl.pallas_call(
        _mm3_body,
        grid=(n_i,),
        in_specs=[
            pl.BlockSpec((NC, bm, 128), lambda i: (0, i, 0)),
            pl.BlockSpec((NC, bm, 128), lambda i: (0, i, 0)),
            pl.BlockSpec((1, bm, 128), lambda i: (0, i, 0)),
            pl.BlockSpec((128, 512), lambda i: (0, 0)),
            pl.BlockSpec((1, 1, 128), lambda i: (0, 0, 0)),
            pl.BlockSpec((bm, 1), lambda i: (i, 0)),
        ],
        out_specs=pl.BlockSpec((bm, 128), lambda i: (i, 0)),
        out_shape=rs((NP, 128)),
    )(z_st, r_st, s_p, Ws2, (bs2 + bn2).reshape(1, 1, 128), inv)

    return out[:N]


# speedup vs baseline: 5.8204x; 1.0021x over previous
import jax
import jax.numpy as jnp
from jax import lax
from jax.experimental import pallas as pl
from jax.experimental.pallas import tpu as pltpu
from jax.experimental.pallas import tpu_sc as plsc

N = 10000
E = 160000
NP = 10240
EP = 163840
K = 128
NC = 2
NS = 16
HALF = NP // 2
AH = HALF + 128
ZPT = AH // NS
WPT = HALF // NS


def _make_agg(two_chunks: bool):
    nc = EP // NS // K
    assert nc % 2 == 0
    nq = 2 if two_chunks else 1

    scratch = [
        pltpu.VMEM((nc, K), jnp.int32),
        pltpu.VMEM((2, nc, K), jnp.int32),
        pltpu.VMEM((K, 128), jnp.float32),
        pltpu.VMEM((K, 128), jnp.float32),
        pltpu.VMEM_SHARED((AH, 128), jnp.float32),
        pltpu.SemaphoreType.DMA,
        pltpu.SemaphoreType.DMA,
    ]
    mesh = plsc.VectorSubcoreMesh(core_axis_name="c", subcore_axis_name="s")

    def body(m_hbm, src_hbm, dst_hbm, zero_hbm, out_hbm,
             idxs, idxd, b0, b1, acc, sem0, sem1):
        c = lax.axis_index("c")
        s = lax.axis_index("s")

        if two_chunks:
            pltpu.sync_copy(src_hbm.at[c * NS + s], idxs)
            pltpu.sync_copy(dst_hbm.at[0].at[s], idxd.at[0])
            pltpu.sync_copy(dst_hbm.at[1].at[s], idxd.at[1])
        else:
            pltpu.sync_copy(src_hbm.at[s], idxs)
            pltpu.sync_copy(dst_hbm.at[c].at[s], idxd.at[0])

        for r in range(2 if two_chunks else 1):
            ixd = idxd.at[r]
            q = c if two_chunks else 0
            h = r if two_chunks else c

            pltpu.sync_copy(zero_hbm.at[pl.ds(s * ZPT, ZPT)],
                            acc.at[pl.ds(s * ZPT, ZPT)])
            plsc.subcore_barrier()

            pltpu.async_copy(m_hbm.at[idxs.at[0]], b0, sem0)

            def step(i, carry):
                j0 = 2 * i
                j1 = 2 * i + 1
                pltpu.async_copy(m_hbm.at[idxs.at[j1]], b1, sem1)
                pltpu.make_async_copy(m_hbm.at[idxs.at[j0]], b0, sem0).wait()
                pltpu.sync_copy(b0, acc.at[ixd.at[j0]], add=True)

                @pl.when(j1 + 1 < nc)
                def _():
                    pltpu.async_copy(m_hbm.at[idxs.at[j1 + 1]], b0, sem0)
                pltpu.make_async_copy(m_hbm.at[idxs.at[j1]], b1, sem1).wait()
                pltpu.sync_copy(b1, acc.at[ixd.at[j1]], add=True)
                return carry

            lax.fori_loop(0, nc // 2, step, 0)
            plsc.subcore_barrier()

            pltpu.sync_copy(
                acc.at[pl.ds(s * WPT, WPT)],
                out_hbm.at[q].at[pl.ds(h * HALF + s * WPT, WPT)])
            if two_chunks and r == 0:
                plsc.subcore_barrier()

    return pl.kernel(body,
                     out_type=jax.ShapeDtypeStruct((nq, NP, 128), jnp.float32),
                     mesh=mesh, scratch_types=tuple(scratch))


def _make_deg():
    nc = EP // NS // K
    scratch = [
        pltpu.VMEM((nc, K), jnp.int32),
        pltpu.VMEM((K, 128), jnp.float32),
        pltpu.VMEM_SHARED((AH, 128), jnp.float32),
    ]
    mesh = plsc.VectorSubcoreMesh(core_axis_name="c", subcore_axis_name="s")

    def body(dst_hbm, ones_hbm, zero_hbm, deg_hbm, idxd, onesb, dacc):
        c = lax.axis_index("c")
        s = lax.axis_index("s")
        pltpu.sync_copy(dst_hbm.at[c].at[s], idxd)
        pltpu.sync_copy(ones_hbm, onesb)
        pltpu.sync_copy(zero_hbm.at[pl.ds(s * ZPT, ZPT)],
                        dacc.at[pl.ds(s * ZPT, ZPT)])
        plsc.subcore_barrier()

        def step(j, carry):
            pltpu.sync_copy(onesb, dacc.at[idxd.at[j]], add=True)
            return carry

        lax.fori_loop(0, nc, step, 0)
        plsc.subcore_barrier()
        pltpu.sync_copy(dacc.at[pl.ds(s * WPT, WPT)],
                        deg_hbm.at[pl.ds(c * HALF + s * WPT, WPT)])

    return pl.kernel(body,
                     out_type=jax.ShapeDtypeStruct((NP, 128), jnp.float32),
                     mesh=mesh, scratch_types=tuple(scratch))


def _dotT(a, w):
    return lax.dot_general(a, w, (((1,), (1,)), ((), ())),
                           preferred_element_type=jnp.float32)


def _selfmm_body(x_ref, w_ref, b_ref, o_ref):
    w = w_ref[...]
    o_ref[0] = (_dotT(x_ref[0], w[:, :128]) + _dotT(x_ref[1], w[:, 128:])
                + b_ref[0])


def _comb1_body(f_ref, s_ref, wn_ref, inv_ref, o_ref):
    wn = wn_ref[...]
    inv = inv_ref[...]
    o = (f_ref[0] + _dotT(s_ref[0] * inv, wn[:, :128])
         + _dotT(s_ref[1] * inv, wn[:, 128:]))
    o_ref[0] = jnp.maximum(o, 0.0)


def _comb2_body(f_ref, s_ref, wn_ref, wpz_ref, wpr_ref, inv_ref,
                z_ref, r_ref, p_ref):
    c = pl.program_id(1)
    wn = wn_ref[...]
    inv = inv_ref[...]
    z = (f_ref[0] + _dotT(s_ref[0] * inv, wn[:, :128])
         + _dotT(s_ref[1] * inv, wn[:, 128:]))
    r = jnp.maximum(z, 0.0)
    z_ref[0] = z
    r_ref[0] = r
    contrib = _dotT(z, wpz_ref[...]) + _dotT(r, wpr_ref[...])

    @pl.when(c == 0)
    def _():
        p_ref[...] = contrib

    @pl.when(c != 0)
    def _():
        p_ref[...] = p_ref[...] + contrib


def _selfmm3_body(z_ref, r_ref, ws_ref, b_ref, o_ref):
    ws = ws_ref[...]
    o_ref[...] = (_dotT(z_ref[0], ws[:, 0:128])
                  + _dotT(z_ref[1], ws[:, 128:256])
                  + _dotT(r_ref[0], ws[:, 256:384])
                  + _dotT(r_ref[1], ws[:, 384:512])
                  + b_ref[0])


def _comb3_body(f_ref, sp_ref, inv_ref, o_ref):
    o_ref[...] = f_ref[...] + sp_ref[0] * inv_ref[...]


def kernel(x, edge_index, Ws0, bs0, Wn0, bn0, Ws1, bs1, Wn1, bn1,
           Ws2, bs2, Wn2, bn2):
    f32 = jnp.float32
    src = edge_index[0].astype(jnp.int32)
    dst = edge_index[1].astype(jnp.int32)

    pad = EP - E
    pad_src = (jnp.arange(pad, dtype=jnp.int32) * 64) % N
    src_p = jnp.concatenate([src, pad_src])
    dst_p = jnp.concatenate([dst, jnp.full((pad,), N, jnp.int32)])

    ept = EP // NS
    nc = ept // K

    trash = HALF + (dst_p % 128)
    dst_loc = []
    for h in range(2):
        lo = h * HALF
        in_h = (dst_p >= lo) & (dst_p < lo + HALF)
        dst_loc.append(jnp.where(in_h, dst_p - lo, trash))
    dst2 = jnp.stack(dst_loc).reshape(2, NS, nc, K)

    src_a = jnp.concatenate([src_p, src_p + NP]).reshape(NC * NS, nc, K)
    src_1 = src_p.reshape(NS, nc, K)

    zeroA = jnp.zeros((AH, 128), f32)
    onesK = jnp.ones((K, 128), f32)

    xp = jnp.zeros((NP, 256), f32).at[:N].set(x)
    x_st = jnp.stack([xp[:, :128], xp[:, 128:]])

    agg2c = _make_agg(True)
    agg1c = _make_agg(False)
    deg_kernel = _make_deg()

    degm = deg_kernel(dst2, onesK, zeroA)
    s_x = agg2c(x_st.reshape(NC * NP, 128), src_a, dst2, zeroA)
    inv = (1.0 / jnp.clip(degm[:, 0], 1.0, None)).reshape(NP, 1)

    bm = 1024
    n_i = NP // bm
    grid2 = (n_i, NC)

    def rs(shape, f=f32):
        return jax.ShapeDtypeStruct(shape, f)

    bspec_st = pl.BlockSpec((NC, bm, 128), lambda i, c: (0, i, 0))
    bspec_out = pl.BlockSpec((1, bm, 128), lambda i, c: (c, i, 0))
    bspec_w = pl.BlockSpec((128, 256), lambda i, c: (c, 0))
    bspec_b = pl.BlockSpec((1, 1, 128), lambda i, c: (c, 0, 0))
    bspec_inv = pl.BlockSpec((bm, 1), lambda i, c: (i, 0))

    def selfmm(xs, W, b2d):
        return pl.pallas_call(
            _selfmm_body, grid=grid2,
            in_specs=[bspec_st, bspec_w, bspec_b],
            out_specs=bspec_out, out_shape=rs((NC, NP, 128)),
        )(xs, W, b2d)

    f1 = selfmm(x_st, Ws0, (bs0 + bn0).reshape(2, 1, 128))

    a_st = pl.pallas_call(
        _comb1_body, grid=grid2,
        in_specs=[bspec_st, bspec_st, bspec_w, bspec_inv],
        out_specs=bspec_out, out_shape=rs((NC, NP, 128)),
    )(f1, s_x, Wn0, inv)

    s_a = agg2c(a_st.reshape(NC * NP, 128), src_a, dst2, zeroA)
    f2 = selfmm(a_st, Ws1, (bs1 + bn1).reshape(2, 1, 128))

    z_st, r_st, p2 = pl.pallas_call(
        _comb2_body, grid=grid2,
        in_specs=[
            bspec_st, bspec_st, bspec_w,
            pl.BlockSpec((128, 128), lambda i, c: (0, c)),
            pl.BlockSpec((128, 128), lambda i, c: (0, 2 + c)),
            bspec_inv,
        ],
        out_specs=[bspec_out, bspec_out,
                   pl.BlockSpec((bm, 128), lambda i, c: (i, 0))],
        out_shape=[rs((NC, NP, 128)), rs((NC, NP, 128)), rs((NP, 128))],
    )(f2, s_a, Wn1, Wn2, Wn2, inv)

    s_p = agg1c(p2, src_1, dst2, zeroA)
    f3 = pl.pallas_call(
        _selfmm3_body, grid=(n_i,),
        in_specs=[
            pl.BlockSpec((NC, bm, 128), lambda i: (0, i, 0)),
            pl.BlockSpec((NC, bm, 128), lambda i: (0, i, 0)),
            pl.BlockSpec((128, 512), lambda i: (0, 0)),
            pl.BlockSpec((1, 1, 128), lambda i: (0, 0, 0)),
        ],
        out_specs=pl.BlockSpec((bm, 128), lambda i: (i, 0)),
        out_shape=rs((NP, 128)),
    )(z_st, r_st, Ws2, (bs2 + bn2).reshape(1, 1, 128))

    out = pl.pallas_call(
        _comb3_body, grid=(n_i,),
        in_specs=[
            pl.BlockSpec((bm, 128), lambda i: (i, 0)),
            pl.BlockSpec((1, bm, 128), lambda i: (0, i, 0)),
            pl.BlockSpec((bm, 1), lambda i: (i, 0)),
        ],
        out_specs=pl.BlockSpec((bm, 128), lambda i: (i, 0)),
        out_shape=rs((NP, 128)),
    )(f3, s_p, inv)

    return out[:N]


# speedup vs baseline: 5.9440x; 1.0212x over previous
import jax
import jax.numpy as jnp
from jax import lax
from jax.experimental import pallas as pl
from jax.experimental.pallas import tpu as pltpu
from jax.experimental.pallas import tpu_sc as plsc

N = 10000
E = 160000
NP = 10240
EP = 163840
K = 128
NC = 2
NS = 16
HALF = NP // 2
AH = HALF + 128
ZPT = AH // NS
WPT = HALF // NS


def _make_agg(two_chunks: bool):
    nc = EP // NS // K
    assert nc % 2 == 0
    nq = 2 if two_chunks else 1

    scratch = [
        pltpu.VMEM((nc, K), jnp.int32),
        pltpu.VMEM((2, nc, K), jnp.int32),
        pltpu.VMEM((K, 128), jnp.float32),
        pltpu.VMEM((K, 128), jnp.float32),
        pltpu.VMEM_SHARED((AH, 128), jnp.float32),
        pltpu.SemaphoreType.DMA,
        pltpu.SemaphoreType.DMA,
    ]
    mesh = plsc.VectorSubcoreMesh(core_axis_name="c", subcore_axis_name="s")

    def body(m_hbm, src_hbm, dst_hbm, zero_hbm, out_hbm,
             idxs, idxd, b0, b1, acc, sem0, sem1):
        c = lax.axis_index("c")
        s = lax.axis_index("s")

        if two_chunks:
            pltpu.sync_copy(src_hbm.at[c * NS + s], idxs)
            pltpu.sync_copy(dst_hbm.at[0].at[s], idxd.at[0])
            pltpu.sync_copy(dst_hbm.at[1].at[s], idxd.at[1])
        else:
            pltpu.sync_copy(src_hbm.at[s], idxs)
            pltpu.sync_copy(dst_hbm.at[c].at[s], idxd.at[0])

        for r in range(2 if two_chunks else 1):
            ixd = idxd.at[r]
            q = c if two_chunks else 0
            h = r if two_chunks else c

            pltpu.sync_copy(zero_hbm.at[pl.ds(s * ZPT, ZPT)],
                            acc.at[pl.ds(s * ZPT, ZPT)])
            plsc.subcore_barrier()

            pltpu.async_copy(m_hbm.at[idxs.at[0]], b0, sem0)

            def step(i, carry):
                j0 = 2 * i
                j1 = 2 * i + 1
                pltpu.async_copy(m_hbm.at[idxs.at[j1]], b1, sem1)
                pltpu.make_async_copy(m_hbm.at[idxs.at[j0]], b0, sem0).wait()
                pltpu.sync_copy(b0, acc.at[ixd.at[j0]], add=True)

                @pl.when(j1 + 1 < nc)
                def _():
                    pltpu.async_copy(m_hbm.at[idxs.at[j1 + 1]], b0, sem0)
                pltpu.make_async_copy(m_hbm.at[idxs.at[j1]], b1, sem1).wait()
                pltpu.sync_copy(b1, acc.at[ixd.at[j1]], add=True)
                return carry

            lax.fori_loop(0, nc // 2, step, 0)
            plsc.subcore_barrier()

            pltpu.sync_copy(
                acc.at[pl.ds(s * WPT, WPT)],
                out_hbm.at[q].at[pl.ds(h * HALF + s * WPT, WPT)])
            if two_chunks and r == 0:
                plsc.subcore_barrier()

    return pl.kernel(body,
                     out_type=jax.ShapeDtypeStruct((nq, NP, 128), jnp.float32),
                     mesh=mesh, scratch_types=tuple(scratch))


def _make_deg():
    nc = EP // NS // K
    scratch = [
        pltpu.VMEM((nc, K), jnp.int32),
        pltpu.VMEM((K, 128), jnp.float32),
        pltpu.VMEM_SHARED((AH, 128), jnp.float32),
    ]
    mesh = plsc.VectorSubcoreMesh(core_axis_name="c", subcore_axis_name="s")

    def body(dst_hbm, ones_hbm, zero_hbm, deg_hbm, idxd, onesb, dacc):
        c = lax.axis_index("c")
        s = lax.axis_index("s")
        pltpu.sync_copy(dst_hbm.at[c].at[s], idxd)
        pltpu.sync_copy(ones_hbm, onesb)
        pltpu.sync_copy(zero_hbm.at[pl.ds(s * ZPT, ZPT)],
                        dacc.at[pl.ds(s * ZPT, ZPT)])
        plsc.subcore_barrier()

        def step(j, carry):
            pltpu.sync_copy(onesb, dacc.at[idxd.at[j]], add=True)
            return carry

        lax.fori_loop(0, nc, step, 0)
        plsc.subcore_barrier()
        pltpu.sync_copy(dacc.at[pl.ds(s * WPT, WPT)],
                        deg_hbm.at[pl.ds(c * HALF + s * WPT, WPT)])

    return pl.kernel(body,
                     out_type=jax.ShapeDtypeStruct((NP, 128), jnp.float32),
                     mesh=mesh, scratch_types=tuple(scratch))


def _dotT(a, w):
    return lax.dot_general(a, w, (((1,), (1,)), ((), ())),
                           preferred_element_type=jnp.float32)


def _mm1_body(x_ref, s_ref, ws_ref, wn_ref, b_ref, inv_ref, o_ref):
    ws = ws_ref[...]
    wn = wn_ref[...]
    inv = inv_ref[...]
    o = (_dotT(x_ref[0], ws[:, :128]) + _dotT(x_ref[1], ws[:, 128:])
         + _dotT(s_ref[0] * inv, wn[:, :128])
         + _dotT(s_ref[1] * inv, wn[:, 128:])
         + b_ref[0])
    o_ref[0] = jnp.maximum(o, 0.0)


def _mm2_body(a_ref, s_ref, ws_ref, wn_ref, b_ref, wpz_ref, wpr_ref, inv_ref,
              z_ref, r_ref, p_ref):
    c = pl.program_id(1)
    ws = ws_ref[...]
    wn = wn_ref[...]
    inv = inv_ref[...]
    z = (_dotT(a_ref[0], ws[:, :128]) + _dotT(a_ref[1], ws[:, 128:])
         + _dotT(s_ref[0] * inv, wn[:, :128])
         + _dotT(s_ref[1] * inv, wn[:, 128:])
         + b_ref[0])
    r = jnp.maximum(z, 0.0)
    z_ref[0] = z
    r_ref[0] = r
    contrib = _dotT(z, wpz_ref[...]) + _dotT(r, wpr_ref[...])

    @pl.when(c == 0)
    def _():
        p_ref[...] = contrib

    @pl.when(c != 0)
    def _():
        p_ref[...] = p_ref[...] + contrib


def _mm3_body(z_ref, r_ref, sp_ref, ws_ref, b_ref, inv_ref, o_ref):
    ws = ws_ref[...]
    inv = inv_ref[...]
    o_ref[...] = (_dotT(z_ref[0], ws[:, 0:128])
                  + _dotT(z_ref[1], ws[:, 128:256])
                  + _dotT(r_ref[0], ws[:, 256:384])
                  + _dotT(r_ref[1], ws[:, 384:512])
                  + sp_ref[0] * inv + b_ref[0])


def kernel(x, edge_index, Ws0, bs0, Wn0, bn0, Ws1, bs1, Wn1, bn1,
           Ws2, bs2, Wn2, bn2):
    f32 = jnp.float32
    src = edge_index[0].astype(jnp.int32)
    dst = edge_index[1].astype(jnp.int32)

    pad = EP - E
    pad_src = (jnp.arange(pad, dtype=jnp.int32) * 64) % N
    src_p = jnp.concatenate([src, pad_src])
    dst_p = jnp.concatenate([dst, jnp.full((pad,), N, jnp.int32)])

    ept = EP // NS
    nc = ept // K

    trash = HALF + (dst_p % 128)
    dst_loc = []
    for h in range(2):
        lo = h * HALF
        in_h = (dst_p >= lo) & (dst_p < lo + HALF)
        dst_loc.append(jnp.where(in_h, dst_p - lo, trash))
    dst2 = jnp.stack(dst_loc).reshape(2, NS, nc, K)

    src_a = jnp.concatenate([src_p, src_p + NP]).reshape(NC * NS, nc, K)
    src_1 = src_p.reshape(NS, nc, K)

    zeroA = jnp.zeros((AH, 128), f32)
    onesK = jnp.ones((K, 128), f32)

    xp = jnp.zeros((NP, 256), f32).at[:N].set(x)
    x_st = jnp.stack([xp[:, :128], xp[:, 128:]])

    agg2c = _make_agg(True)
    agg1c = _make_agg(False)
    deg_kernel = _make_deg()

    degm = deg_kernel(dst2, onesK, zeroA)
    s_x = agg2c(x_st.reshape(NC * NP, 128), src_a, dst2, zeroA)
    inv = (1.0 / jnp.clip(degm[:, 0], 1.0, None)).reshape(NP, 1)

    bm = 2048
    n_i = NP // bm
    grid2 = (n_i, NC)

    def rs(shape, f=f32):
        return jax.ShapeDtypeStruct(shape, f)

    a_st = pl.pallas_call(
        _mm1_body,
        grid=grid2,
        in_specs=[
            pl.BlockSpec((NC, bm, 128), lambda i, c: (0, i, 0)),
            pl.BlockSpec((NC, bm, 128), lambda i, c: (0, i, 0)),
            pl.BlockSpec((128, 256), lambda i, c: (c, 0)),
            pl.BlockSpec((128, 256), lambda i, c: (c, 0)),
            pl.BlockSpec((1, 1, 128), lambda i, c: (c, 0, 0)),
            pl.BlockSpec((bm, 1), lambda i, c: (i, 0)),
        ],
        out_specs=pl.BlockSpec((1, bm, 128), lambda i, c: (c, i, 0)),
        out_shape=rs((NC, NP, 128)),
    )(x_st, s_x, Ws0, Wn0, (bs0 + bn0).reshape(2, 1, 128), inv)

    s_a = agg2c(a_st.reshape(NC * NP, 128), src_a, dst2, zeroA)

    z_st, r_st, p2 = pl.pallas_call(
        _mm2_body,
        grid=grid2,
        in_specs=[
            pl.BlockSpec((NC, bm, 128), lambda i, c: (0, i, 0)),
            pl.BlockSpec((NC, bm, 128), lambda i, c: (0, i, 0)),
            pl.BlockSpec((128, 256), lambda i, c: (c, 0)),
            pl.BlockSpec((128, 256), lambda i, c: (c, 0)),
            pl.BlockSpec((1, 1, 128), lambda i, c: (c, 0, 0)),
            pl.BlockSpec((128, 128), lambda i, c: (0, c)),
            pl.BlockSpec((128, 128), lambda i, c: (0, 2 + c)),
            pl.BlockSpec((bm, 1), lambda i, c: (i, 0)),
        ],
        out_specs=[
            pl.BlockSpec((1, bm, 128), lambda i, c: (c, i, 0)),
            pl.BlockSpec((1, bm, 128), lambda i, c: (c, i, 0)),
            pl.BlockSpec((bm, 128), lambda i, c: (i, 0)),
        ],
        out_shape=[rs((NC, NP, 128)), rs((NC, NP, 128)), rs((NP, 128))],
    )(a_st, s_a, Ws1, Wn1, (bs1 + bn1).reshape(2, 1, 128), Wn2, Wn2, inv)

    s_p = agg1c(p2, src_1, dst2, zeroA)

    out = pl.pallas_call(
        _mm3_body,
        grid=(n_i,),
        in_specs=[
            pl.BlockSpec((NC, bm, 128), lambda i: (0, i, 0)),
            pl.BlockSpec((NC, bm, 128), lambda i: (0, i, 0)),
            pl.BlockSpec((1, bm, 128), lambda i: (0, i, 0)),
            pl.BlockSpec((128, 512), lambda i: (0, 0)),
            pl.BlockSpec((1, 1, 128), lambda i: (0, 0, 0)),
            pl.BlockSpec((bm, 1), lambda i: (i, 0)),
        ],
        out_specs=pl.BlockSpec((bm, 128), lambda i: (i, 0)),
        out_shape=rs((NP, 128)),
    )(z_st, r_st, s_p, Ws2, (bs2 + bn2).reshape(1, 1, 128), inv)

    return out[:N]


# speedup vs baseline: 5.9481x; 1.0007x over previous
import jax
import jax.numpy as jnp
from jax import lax
from jax.experimental import pallas as pl
from jax.experimental.pallas import tpu as pltpu
from jax.experimental.pallas import tpu_sc as plsc

N = 10000
E = 160000
NP = 10240
EP = 163840
K = 128
NC = 2
NS = 16
HALF = NP // 2
AH = HALF + 128
ZPT = AH // NS
WPT = HALF // NS


def _make_agg(two_chunks: bool):
    nc = EP // NS // K
    assert nc % 2 == 0
    nq = 2 if two_chunks else 1

    scratch = [
        pltpu.VMEM((nc, K), jnp.int32),
        pltpu.VMEM((2, nc, K), jnp.int32),
        pltpu.VMEM((K, 128), jnp.float32),
        pltpu.VMEM((K, 128), jnp.float32),
        pltpu.VMEM_SHARED((AH, 128), jnp.float32),
        pltpu.SemaphoreType.DMA,
        pltpu.SemaphoreType.DMA,
    ]
    mesh = plsc.VectorSubcoreMesh(core_axis_name="c", subcore_axis_name="s")

    def body(m_hbm, src_hbm, dst_hbm, zero_hbm, out_hbm,
             idxs, idxd, b0, b1, acc, sem0, sem1):
        c = lax.axis_index("c")
        s = lax.axis_index("s")

        if two_chunks:
            pltpu.sync_copy(src_hbm.at[c * NS + s], idxs)
            pltpu.sync_copy(dst_hbm.at[0].at[s], idxd.at[0])
            pltpu.sync_copy(dst_hbm.at[1].at[s], idxd.at[1])
        else:
            pltpu.sync_copy(src_hbm.at[s], idxs)
            pltpu.sync_copy(dst_hbm.at[c].at[s], idxd.at[0])

        for r in range(2 if two_chunks else 1):
            ixd = idxd.at[r]
            q = c if two_chunks else 0
            h = r if two_chunks else c

            pltpu.sync_copy(zero_hbm.at[pl.ds(s * ZPT, ZPT)],
                            acc.at[pl.ds(s * ZPT, ZPT)])
            plsc.subcore_barrier()

            pltpu.async_copy(m_hbm.at[idxs.at[0]], b0, sem0)

            def step(i, carry):
                j0 = 2 * i
                j1 = 2 * i + 1
                pltpu.async_copy(m_hbm.at[idxs.at[j1]], b1, sem1)
                pltpu.make_async_copy(m_hbm.at[idxs.at[j0]], b0, sem0).wait()
                pltpu.sync_copy(b0, acc.at[ixd.at[j0]], add=True)

                @pl.when(j1 + 1 < nc)
                def _():
                    pltpu.async_copy(m_hbm.at[idxs.at[j1 + 1]], b0, sem0)
                pltpu.make_async_copy(m_hbm.at[idxs.at[j1]], b1, sem1).wait()
                pltpu.sync_copy(b1, acc.at[ixd.at[j1]], add=True)
                return carry

            lax.fori_loop(0, nc // 2, step, 0)
            plsc.subcore_barrier()

            pltpu.sync_copy(
                acc.at[pl.ds(s * WPT, WPT)],
                out_hbm.at[q].at[pl.ds(h * HALF + s * WPT, WPT)])
            if two_chunks and r == 0:
                plsc.subcore_barrier()

    return pl.kernel(body,
                     out_type=jax.ShapeDtypeStruct((nq, NP, 128), jnp.float32),
                     mesh=mesh, scratch_types=tuple(scratch))


def _make_deg():
    nc = EP // NS // K
    scratch = [
        pltpu.VMEM((nc, K), jnp.int32),
        pltpu.VMEM((K, 128), jnp.float32),
        pltpu.VMEM_SHARED((AH, 128), jnp.float32),
        pltpu.SemaphoreType.DMA,
        pltpu.SemaphoreType.DMA,
    ]
    mesh = plsc.VectorSubcoreMesh(core_axis_name="c", subcore_axis_name="s")

    def body(dst_hbm, ones_hbm, zero_hbm, deg_hbm, idxd, onesb, dacc,
             sem0, sem1):
        c = lax.axis_index("c")
        s = lax.axis_index("s")
        pltpu.sync_copy(dst_hbm.at[c].at[s], idxd)
        pltpu.sync_copy(ones_hbm, onesb)
        pltpu.sync_copy(zero_hbm.at[pl.ds(s * ZPT, ZPT)],
                        dacc.at[pl.ds(s * ZPT, ZPT)])
        plsc.subcore_barrier()

        pltpu.async_copy(onesb, dacc.at[idxd.at[0]], sem0, add=True)

        def step(i, carry):
            j0 = 2 * i
            j1 = 2 * i + 1
            pltpu.async_copy(onesb, dacc.at[idxd.at[j1]], sem1, add=True)
            pltpu.make_async_copy(onesb, dacc.at[idxd.at[j0]], sem0).wait()

            @pl.when(j1 + 1 < nc)
            def _():
                pltpu.async_copy(onesb, dacc.at[idxd.at[j1 + 1]], sem0,
                                 add=True)
            pltpu.make_async_copy(onesb, dacc.at[idxd.at[j1]], sem1).wait()
            return carry

        lax.fori_loop(0, nc // 2, step, 0)
        plsc.subcore_barrier()
        pltpu.sync_copy(dacc.at[pl.ds(s * WPT, WPT)],
                        deg_hbm.at[pl.ds(c * HALF + s * WPT, WPT)])

    return pl.kernel(body,
                     out_type=jax.ShapeDtypeStruct((NP, 128), jnp.float32),
                     mesh=mesh, scratch_types=tuple(scratch))


def _dotT(a, w):
    return lax.dot_general(a, w, (((1,), (1,)), ((), ())),
                           preferred_element_type=jnp.float32)


def _mm1_body(x_ref, s_ref, ws_ref, wn_ref, b_ref, inv_ref, o_ref):
    ws = ws_ref[...]
    wn = wn_ref[...]
    inv = inv_ref[...]
    o = (_dotT(x_ref[0], ws[:, :128]) + _dotT(x_ref[1], ws[:, 128:])
         + _dotT(s_ref[0] * inv, wn[:, :128])
         + _dotT(s_ref[1] * inv, wn[:, 128:])
         + b_ref[0])
    o_ref[0] = jnp.maximum(o, 0.0)


def _mm2_body(a_ref, s_ref, ws_ref, wn_ref, b_ref, wpz_ref, wpr_ref, inv_ref,
              z_ref, r_ref, p_ref):
    c = pl.program_id(1)
    ws = ws_ref[...]
    wn = wn_ref[...]
    inv = inv_ref[...]
    z = (_dotT(a_ref[0], ws[:, :128]) + _dotT(a_ref[1], ws[:, 128:])
         + _dotT(s_ref[0] * inv, wn[:, :128])
         + _dotT(s_ref[1] * inv, wn[:, 128:])
         + b_ref[0])
    r = jnp.maximum(z, 0.0)
    z_ref[0] = z
    r_ref[0] = r
    contrib = _dotT(z, wpz_ref[...]) + _dotT(r, wpr_ref[...])

    @pl.when(c == 0)
    def _():
        p_ref[...] = contrib

    @pl.when(c != 0)
    def _():
        p_ref[...] = p_ref[...] + contrib


def _mm3_body(z_ref, r_ref, sp_ref, ws_ref, b_ref, inv_ref, o_ref):
    ws = ws_ref[...]
    inv = inv_ref[...]
    o_ref[...] = (_dotT(z_ref[0], ws[:, 0:128])
                  + _dotT(z_ref[1], ws[:, 128:256])
                  + _dotT(r_ref[0], ws[:, 256:384])
                  + _dotT(r_ref[1], ws[:, 384:512])
                  + sp_ref[0] * inv + b_ref[0])


def kernel(x, edge_index, Ws0, bs0, Wn0, bn0, Ws1, bs1, Wn1, bn1,
           Ws2, bs2, Wn2, bn2):
    f32 = jnp.float32
    src = edge_index[0].astype(jnp.int32)
    dst = edge_index[1].astype(jnp.int32)

    pad = EP - E
    pad_src = (jnp.arange(pad, dtype=jnp.int32) * 64) % N
    src_p = jnp.concatenate([src, pad_src])
    dst_p = jnp.concatenate([dst, jnp.full((pad,), N, jnp.int32)])

    ept = EP // NS
    nc = ept // K

    trash = HALF + (dst_p % 128)
    dst_loc = []
    for h in range(2):
        lo = h * HALF
        in_h = (dst_p >= lo) & (dst_p < lo + HALF)
        dst_loc.append(jnp.where(in_h, dst_p - lo, trash))
    dst2 = jnp.stack(dst_loc).reshape(2, NS, nc, K)

    src_a = jnp.concatenate([src_p, src_p + NP]).reshape(NC * NS, nc, K)
    src_1 = src_p.reshape(NS, nc, K)

    zeroA = jnp.zeros((AH, 128), f32)
    onesK = jnp.ones((K, 128), f32)

    xp = jnp.zeros((NP, 256), f32).at[:N].set(x)
    x_st = jnp.stack([xp[:, :128], xp[:, 128:]])

    agg2c = _make_agg(True)
    agg1c = _make_agg(False)
    deg_kernel = _make_deg()

    degm = deg_kernel(dst2, onesK, zeroA)
    s_x = agg2c(x_st.reshape(NC * NP, 128), src_a, dst2, zeroA)
    inv = (1.0 / jnp.clip(degm[:, 0], 1.0, None)).reshape(NP, 1)

    bm = 2048
    n_i = NP // bm
    grid2 = (n_i, NC)

    def rs(shape, f=f32):
        return jax.ShapeDtypeStruct(shape, f)

    a_st = pl.pallas_call(
        _mm1_body,
        grid=grid2,
        in_specs=[
            pl.BlockSpec((NC, bm, 128), lambda i, c: (0, i, 0)),
            pl.BlockSpec((NC, bm, 128), lambda i, c: (0, i, 0)),
            pl.BlockSpec((128, 256), lambda i, c: (c, 0)),
            pl.BlockSpec((128, 256), lambda i, c: (c, 0)),
            pl.BlockSpec((1, 1, 128), lambda i, c: (c, 0, 0)),
            pl.BlockSpec((bm, 1), lambda i, c: (i, 0)),
        ],
        out_specs=pl.BlockSpec((1, bm, 128), lambda i, c: (c, i, 0)),
        out_shape=rs((NC, NP, 128)),
    )(x_st, s_x, Ws0, Wn0, (bs0 + bn0).reshape(2, 1, 128), inv)

    s_a = agg2c(a_st.reshape(NC * NP, 128), src_a, dst2, zeroA)

    z_st, r_st, p2 = pl.pallas_call(
        _mm2_body,
        grid=grid2,
        in_specs=[
            pl.BlockSpec((NC, bm, 128), lambda i, c: (0, i, 0)),
            pl.BlockSpec((NC, bm, 128), lambda i, c: (0, i, 0)),
            pl.BlockSpec((128, 256), lambda i, c: (c, 0)),
            pl.BlockSpec((128, 256), lambda i, c: (c, 0)),
            pl.BlockSpec((1, 1, 128), lambda i, c: (c, 0, 0)),
            pl.BlockSpec((128, 128), lambda i, c: (0, c)),
            pl.BlockSpec((128, 128), lambda i, c: (0, 2 + c)),
            pl.BlockSpec((bm, 1), lambda i, c: (i, 0)),
        ],
        out_specs=[
            pl.BlockSpec((1, bm, 128), lambda i, c: (c, i, 0)),
            pl.BlockSpec((1, bm, 128), lambda i, c: (c, i, 0)),
            pl.BlockSpec((bm, 128), lambda i, c: (i, 0)),
        ],
        out_shape=[rs((NC, NP, 128)), rs((NC, NP, 128)), rs((NP, 128))],
    )(a_st, s_a, Ws1, Wn1, (bs1 + bn1).reshape(2, 1, 128), Wn2, Wn2, inv)

    s_p = agg1c(p2, src_1, dst2, zeroA)

    out = pl.pallas_call(
        _mm3_body,
        grid=(n_i,),
        in_specs=[
            pl.BlockSpec((NC, bm, 128), lambda i: (0, i, 0)),
            pl.BlockSpec((NC, bm, 128), lambda i: (0, i, 0)),
            pl.BlockSpec((1, bm, 128), lambda i: (0, i, 0)),
            pl.BlockSpec((128, 512), lambda i: (0, 0)),
            pl.BlockSpec((1, 1, 128), lambda i: (0, 0, 0)),
            pl.BlockSpec((bm, 1), lambda i: (i, 0)),
        ],
        out_specs=pl.BlockSpec((bm, 128), lambda i: (i, 0)),
        out_shape=rs((NP, 128)),
    )(z_st, r_st, s_p, Ws2, (bs2 + bn2).reshape(1, 1, 128), inv)

    return out[:N]
